# baseline (speedup 1.0000x reference)
import jax
import jax.numpy as jnp
from jax import lax
from jax.experimental import pallas as pl
from jax.experimental.pallas import tpu as pltpu
from jax.experimental.pallas import tpu_sc as plsc

T = 128
K = 8
H = 4096
EPS = 1e-6
NC, NS = 2, 16
NW = NC * NS
TPW = T // NW
L = 16
NCHUNK = H // L


def _rsqrt_vec(x):
    xi = lax.bitcast_convert_type(x, jnp.int32)
    yi = jnp.int32(0x5F3759DF) - (xi >> 1)
    y = lax.bitcast_convert_type(yi, jnp.float32)
    for _ in range(3):
        y = y * (1.5 - 0.5 * x * y * y)
    return y


def _tec_kernel(inp_hbm, idx_hbm, scale_hbm, res_hbm, sh_hbm, w_hbm,
                hid_out, resout_out,
                idx_v, scale_v, g_v, res_v, sh_v, w_v, outres_v, hid_v, sem):
    wid = lax.axis_index("s") * NC + lax.axis_index("c")
    base = wid * TPW

    pltpu.sync_copy(w_hbm, w_v)
    pltpu.sync_copy(idx_hbm.at[pl.ds(base, TPW)], idx_v)
    pltpu.sync_copy(scale_hbm.at[pl.ds(base * K, TPW * K)], scale_v)
    pltpu.sync_copy(res_hbm.at[pl.ds(base, TPW)], res_v)
    pltpu.sync_copy(sh_hbm.at[pl.ds(base, TPW)], sh_v)

    for t in range(TPW):
        pltpu.async_copy(inp_hbm.at[idx_v.at[t]], g_v, sem).wait()

        svec = scale_v[pl.ds((t // 2) * L, L)]
        s = [svec[(t % 2) * K + kk] for kk in range(K)]

        def chunk_body(c, ssq, t=t, s=s):
            b = c * L
            acc = g_v[0, pl.ds(b, L)] * s[0]
            for kk in range(1, K):
                acc = acc + g_v[kk, pl.ds(b, L)] * s[kk]
            acc = acc + res_v[t, pl.ds(b, L)] + sh_v[t, pl.ds(b, L)]
            outres_v[t, pl.ds(b, L)] = acc
            return ssq + acc * acc

        ssq = lax.fori_loop(0, NCHUNK, chunk_body,
                            jnp.zeros((L,), jnp.float32))
        tot = ssq[0]
        for lane in range(1, L):
            tot = tot + ssq[lane]
        rs = _rsqrt_vec(jnp.full((L,), tot * (1.0 / H) + EPS, jnp.float32))

        def norm_body(c, carry, t=t, rs=rs):
            b = c * L
            hid_v[t, pl.ds(b, L)] = (outres_v[t, pl.ds(b, L)] * rs
                                     * w_v[pl.ds(b, L)])
            return carry

        lax.fori_loop(0, NCHUNK, norm_body, 0)

    pltpu.sync_copy(outres_v, resout_out.at[pl.ds(base, TPW)])
    pltpu.sync_copy(hid_v, hid_out.at[pl.ds(base, TPW)])


_moe_finalize = pl.kernel(
    _tec_kernel,
    out_type=(jax.ShapeDtypeStruct((T, H), jnp.float32),
              jax.ShapeDtypeStruct((T, H), jnp.float32)),
    mesh=plsc.VectorSubcoreMesh(core_axis_name="c", subcore_axis_name="s"),
    scratch_types=[
        pltpu.VMEM((TPW, K), jnp.int32),
        pltpu.VMEM((TPW * K,), jnp.float32),
        pltpu.VMEM((K, H), jnp.float32),
        pltpu.VMEM((TPW, H), jnp.float32),
        pltpu.VMEM((TPW, H), jnp.float32),
        pltpu.VMEM((H,), jnp.float32),
        pltpu.VMEM((TPW, H), jnp.float32),
        pltpu.VMEM((TPW, H), jnp.float32),
        pltpu.SemaphoreType.DMA,
    ],
)


def kernel(input, residual, norm_weight, expanded_idx_to_permuted_idx,
           shared_expert_output, expert_scale_factor):
    hid, outres = _moe_finalize(input, expanded_idx_to_permuted_idx,
                                expert_scale_factor.reshape(T * K), residual,
                                shared_expert_output, norm_weight)
    return (hid, outres)


# speedup vs baseline: 1.2498x; 1.2498x over previous
import jax
import jax.numpy as jnp
from jax import lax
from jax.experimental import pallas as pl
from jax.experimental.pallas import tpu as pltpu
from jax.experimental.pallas import tpu_sc as plsc

T = 128
K = 8
H = 4096
EPS = 1e-6
NC, NS = 2, 16
NW = NC * NS
TPW = T // NW
L = 16
NCHUNK = H // L


def _rsqrt_vec(x):
    xi = lax.bitcast_convert_type(x, jnp.int32)
    yi = jnp.int32(0x5F3759DF) - (xi >> 1)
    y = lax.bitcast_convert_type(yi, jnp.float32)
    for _ in range(3):
        y = y * (1.5 - 0.5 * x * y * y)
    return y


def _tec_kernel(inp_hbm, idx_hbm, scale_hbm, res_hbm, sh_hbm, w_hbm,
                hid_out, resout_out,
                idx_v, scale_v, g_v, res_v, sh_v, w_v, outres_v, hid_v,
                gsem0, gsem1, ldsem, osem0, osem1, hsem0, hsem1):
    wid = lax.axis_index("s") * NC + lax.axis_index("c")
    base = wid * TPW

    gsem = (gsem0, gsem1)
    osem = (osem0, osem1)
    hsem = (hsem0, hsem1)

    pltpu.sync_copy(idx_hbm.at[pl.ds(base, TPW)], idx_v)
    pltpu.sync_copy(scale_hbm.at[pl.ds(base * K, TPW * K)], scale_v)

    gather = [None, None]
    gather[0] = pltpu.async_copy(inp_hbm.at[idx_v.at[0]], g_v.at[0], gsem[0])
    res_cp = pltpu.async_copy(res_hbm.at[pl.ds(base, TPW)], res_v, ldsem)
    sh_cp = pltpu.async_copy(sh_hbm.at[pl.ds(base, TPW)], sh_v, ldsem)
    w_cp = pltpu.async_copy(w_hbm, w_v, ldsem)

    out_pend = [None, None]

    for t in range(TPW):
        b = t % 2
        gather[b].wait()
        if t + 1 < TPW:
            nb = (t + 1) % 2
            gather[nb] = pltpu.async_copy(
                inp_hbm.at[idx_v.at[t + 1]], g_v.at[nb], gsem[nb])
        if t == 0:
            res_cp.wait()
            sh_cp.wait()
            w_cp.wait()
        if out_pend[b] is not None:
            out_pend[b][0].wait()
            out_pend[b][1].wait()

        svec = scale_v[pl.ds((t // 2) * L, L)]
        s = [svec[(t % 2) * K + kk] for kk in range(K)]

        @plsc.parallel_loop(0, NCHUNK, unroll=4,
                            carry=jnp.zeros((L,), jnp.float32))
        def ssq(c, acc_ssq, t=t, b=b, s=s):
            bb = c * L
            acc = g_v[b, 0, pl.ds(bb, L)] * s[0]
            for kk in range(1, K):
                acc = acc + g_v[b, kk, pl.ds(bb, L)] * s[kk]
            acc = acc + res_v[t, pl.ds(bb, L)] + sh_v[t, pl.ds(bb, L)]
            outres_v[b, 0, pl.ds(bb, L)] = acc
            return acc_ssq + acc * acc

        tot = ssq[0]
        for lane in range(1, L):
            tot = tot + ssq[lane]
        rs = _rsqrt_vec(jnp.full((L,), tot * (1.0 / H) + EPS, jnp.float32))

        @plsc.parallel_loop(0, NCHUNK, unroll=4)
        def _(c, b=b, rs=rs):
            bb = c * L
            hid_v[b, 0, pl.ds(bb, L)] = (outres_v[b, 0, pl.ds(bb, L)] * rs
                                         * w_v[pl.ds(bb, L)])

        out_pend[b] = (
            pltpu.async_copy(outres_v.at[b],
                             resout_out.at[pl.ds(base + t, 1)], osem[b]),
            pltpu.async_copy(hid_v.at[b],
                             hid_out.at[pl.ds(base + t, 1)], hsem[b]),
        )

    for b in range(2):
        out_pend[b][0].wait()
        out_pend[b][1].wait()


_moe_finalize = pl.kernel(
    _tec_kernel,
    out_type=(jax.ShapeDtypeStruct((T, H), jnp.float32),
              jax.ShapeDtypeStruct((T, H), jnp.float32)),
    mesh=plsc.VectorSubcoreMesh(core_axis_name="c", subcore_axis_name="s"),
    scratch_types=[
        pltpu.VMEM((TPW, K), jnp.int32),
        pltpu.VMEM((TPW * K,), jnp.float32),
        pltpu.VMEM((2, K, H), jnp.float32),
        pltpu.VMEM((TPW, H), jnp.float32),
        pltpu.VMEM((TPW, H), jnp.float32),
        pltpu.VMEM((H,), jnp.float32),
        pltpu.VMEM((2, 1, H), jnp.float32),
        pltpu.VMEM((2, 1, H), jnp.float32),
        pltpu.SemaphoreType.DMA,
        pltpu.SemaphoreType.DMA,
        pltpu.SemaphoreType.DMA,
        pltpu.SemaphoreType.DMA,
        pltpu.SemaphoreType.DMA,
        pltpu.SemaphoreType.DMA,
        pltpu.SemaphoreType.DMA,
    ],
)


def kernel(input, residual, norm_weight, expanded_idx_to_permuted_idx,
           shared_expert_output, expert_scale_factor):
    hid, outres = _moe_finalize(input, expanded_idx_to_permuted_idx,
                                expert_scale_factor.reshape(T * K), residual,
                                shared_expert_output, norm_weight)
    return (hid, outres)
